# baseline (speedup 1.0000x reference)
import functools

import jax
import jax.numpy as jnp
from jax import lax
from jax.experimental import pallas as pl
from jax.experimental.pallas import tpu as pltpu
from jax.experimental.pallas import tpu_sc as plsc

_LANES = 16
_CHUNK = 32


def kernel(inputs, pos_table):
    batch, seq, dim = inputs.shape
    info = plsc.get_sparse_core_info()
    nw = info.num_cores * info.num_subcores
    seq_per_w = seq // nw
    n_chunks = seq_per_w // _CHUNK
    n_steps = n_chunks * batch
    mesh = plsc.VectorSubcoreMesh(core_axis_name="c", subcore_axis_name="s")

    @functools.partial(
        pl.kernel,
        mesh=mesh,
        out_type=jax.ShapeDtypeStruct((batch, seq, dim), jnp.float32),
        scratch_types=[
            pltpu.VMEM((_CHUNK, dim), jnp.float32),
            pltpu.VMEM((_CHUNK, dim), jnp.float32),
            pltpu.VMEM((_CHUNK, dim), jnp.float32),
            pltpu.VMEM((_CHUNK,), jnp.int32),
            pltpu.SemaphoreType.DMA,
            pltpu.SemaphoreType.DMA,
            pltpu.SemaphoreType.DMA,
            pltpu.SemaphoreType.DMA,
            pltpu.SemaphoreType.DMA,
        ],
    )
    def sc_kernel(in_hbm, pos_hbm, out_hbm, buf0, buf1, pos_v, idx_v,
                  s_in0, s_in1, s_out0, s_out1, s_pos):
        wid = lax.axis_index("s") * info.num_cores + lax.axis_index("c")
        seq0 = wid * seq_per_w
        bufs = (buf0, buf1)
        in_sems = (s_in0, s_in1)
        out_sems = (s_out0, s_out1)
        steps = [(ci, b) for ci in range(n_chunks) for b in range(batch)]

        for v0 in range(0, _CHUNK, _LANES):
            idx_v[pl.ds(v0, _LANES)] = lax.iota(jnp.int32, _LANES) + v0

        pos_h = pltpu.async_copy(
            pos_hbm.at[pl.ds(seq0, _CHUNK), :], pos_v, s_pos)
        in_h = {0: pltpu.async_copy(
            in_hbm.at[0, pl.ds(seq0, _CHUNK), :], bufs[0], in_sems[0])}
        out_h = {}

        for s, (ci, b) in enumerate(steps):
            row0 = seq0 + ci * _CHUNK
            p = s % 2
            if s + 1 < n_steps:
                ci2, b2 = steps[s + 1]
                if s - 1 >= 0:
                    out_h[s - 1].wait()
                in_h[s + 1] = pltpu.async_copy(
                    in_hbm.at[b2, pl.ds(seq0 + ci2 * _CHUNK, _CHUNK), :],
                    bufs[(s + 1) % 2], in_sems[(s + 1) % 2])
            if b == 0:
                pos_h.wait()
            in_h[s].wait()

            buf = bufs[p]

            def vec_body(v, _, buf=buf):
                sl = pl.ds(v * _LANES, _LANES)
                for r in range(_CHUNK):
                    plsc.addupdate(buf.at[r, sl], pos_v[r, sl])
                return 0

            lax.fori_loop(0, dim // _LANES, vec_body, 0)

            if b == batch - 1 and ci + 1 < n_chunks:
                pos_h = pltpu.async_copy(
                    pos_hbm.at[pl.ds(seq0 + (ci + 1) * _CHUNK, _CHUNK), :],
                    pos_v, s_pos)
            out_h[s] = pltpu.async_copy(
                buf, out_hbm.at[b, pl.ds(row0, _CHUNK), :], out_sems[p])

        out_h[n_steps - 2].wait()
        out_h[n_steps - 1].wait()

    return sc_kernel(inputs, pos_table)


# speedup vs baseline: 1.7172x; 1.7172x over previous
import functools

import jax
import jax.numpy as jnp
from jax import lax
from jax.experimental import pallas as pl
from jax.experimental.pallas import tpu as pltpu
from jax.experimental.pallas import tpu_sc as plsc

_LANES = 16
_CHUNK = 32


def kernel(inputs, pos_table):
    batch, seq, dim = inputs.shape
    info = plsc.get_sparse_core_info()
    nw = info.num_cores * info.num_subcores
    seq_per_w = seq // nw
    n_chunks = seq_per_w // _CHUNK
    n_steps = n_chunks * batch
    mesh = plsc.VectorSubcoreMesh(core_axis_name="c", subcore_axis_name="s")

    @functools.partial(
        pl.kernel,
        mesh=mesh,
        out_type=jax.ShapeDtypeStruct((batch, seq, dim), jnp.float32),
        scratch_types=[
            pltpu.VMEM((_CHUNK, dim), jnp.float32),
            pltpu.VMEM((_CHUNK, dim), jnp.float32),
            pltpu.VMEM((_CHUNK, dim), jnp.float32),
            pltpu.VMEM((_CHUNK,), jnp.int32),
            pltpu.SemaphoreType.DMA,
            pltpu.SemaphoreType.DMA,
            pltpu.SemaphoreType.DMA,
            pltpu.SemaphoreType.DMA,
            pltpu.SemaphoreType.DMA,
        ],
    )
    def sc_kernel(in_hbm, pos_hbm, out_hbm, buf0, buf1, pos_v, idx_v,
                  s_in0, s_in1, s_out0, s_out1, s_pos):
        wid = lax.axis_index("s") * info.num_cores + lax.axis_index("c")
        seq0 = wid * seq_per_w
        bufs = (buf0, buf1)
        in_sems = (s_in0, s_in1)
        out_sems = (s_out0, s_out1)
        steps = [(ci, b) for ci in range(n_chunks) for b in range(batch)]

        for v0 in range(0, _CHUNK, _LANES):
            idx_v[pl.ds(v0, _LANES)] = lax.iota(jnp.int32, _LANES) + v0

        pos_h = pltpu.async_copy(
            pos_hbm.at[pl.ds(seq0, _CHUNK), :], pos_v, s_pos)
        in_h = {0: pltpu.async_copy(
            in_hbm.at[0, pl.ds(seq0, _CHUNK), :], bufs[0], in_sems[0])}
        out_h = {}

        for s, (ci, b) in enumerate(steps):
            row0 = seq0 + ci * _CHUNK
            p = s % 2
            if s + 1 < n_steps:
                ci2, b2 = steps[s + 1]
                if s - 1 >= 0:
                    out_h[s - 1].wait()
                in_h[s + 1] = pltpu.async_copy(
                    in_hbm.at[b2, pl.ds(seq0 + ci2 * _CHUNK, _CHUNK), :],
                    bufs[(s + 1) % 2], in_sems[(s + 1) % 2])
            if b == 0:
                pos_h.wait()
            in_h[s].wait()

            buf = bufs[p]

            @plsc.parallel_loop(0, _CHUNK, 1, unroll=1)
            def row_body(r, buf=buf):
                for v in range(dim // _LANES):
                    sl = pl.ds(v * _LANES, _LANES)
                    buf[r, sl] = buf[r, sl] + pos_v[r, sl]

            if b == batch - 1 and ci + 1 < n_chunks:
                pos_h = pltpu.async_copy(
                    pos_hbm.at[pl.ds(seq0 + (ci + 1) * _CHUNK, _CHUNK), :],
                    pos_v, s_pos)
            out_h[s] = pltpu.async_copy(
                buf, out_hbm.at[b, pl.ds(row0, _CHUNK), :], out_sems[p])

        out_h[n_steps - 2].wait()
        out_h[n_steps - 1].wait()

    return sc_kernel(inputs, pos_table)


# speedup vs baseline: 2.2167x; 1.2909x over previous
import functools

import jax
import jax.numpy as jnp
from jax import lax
from jax.experimental import pallas as pl
from jax.experimental.pallas import tpu as pltpu
from jax.experimental.pallas import tpu_sc as plsc

_LANES = 16
_CHUNK = 8
_NSETS = 3


def kernel(inputs, pos_table):
    batch, seq, dim = inputs.shape
    info = plsc.get_sparse_core_info()
    nw = info.num_cores * info.num_subcores
    seq_per_w = seq // nw
    n_chunks = seq_per_w // _CHUNK
    n_vecs = dim // _LANES
    mesh = plsc.VectorSubcoreMesh(core_axis_name="c", subcore_axis_name="s")

    scratch = (
        [pltpu.VMEM((_CHUNK, dim), jnp.float32) for _ in range(_NSETS * batch)]
        + [pltpu.VMEM((_CHUNK, dim), jnp.float32) for _ in range(2)]
        + [pltpu.SemaphoreType.DMA for _ in range(_NSETS * 2 + 2)]
    )

    @functools.partial(
        pl.kernel,
        mesh=mesh,
        out_type=jax.ShapeDtypeStruct((batch, seq, dim), jnp.float32),
        scratch_types=scratch,
    )
    def sc_kernel(in_hbm, pos_hbm, out_hbm, *refs):
        bufs = [refs[r * batch:(r + 1) * batch] for r in range(_NSETS)]
        pos_bufs = refs[_NSETS * batch:_NSETS * batch + 2]
        sems = refs[_NSETS * batch + 2:]
        in_sems = sems[:_NSETS]
        out_sems = sems[_NSETS:2 * _NSETS]
        pos_sems = sems[2 * _NSETS:]

        wid = lax.axis_index("s") * info.num_cores + lax.axis_index("c")
        seq0 = wid * seq_per_w

        def issue_in(c):
            r = c % _NSETS
            row0 = seq0 + c * _CHUNK
            hs = [
                pltpu.async_copy(
                    in_hbm.at[b, pl.ds(row0, _CHUNK), :], bufs[r][b],
                    in_sems[r])
                for b in range(batch)
            ]
            hs.append(pltpu.async_copy(
                pos_hbm.at[pl.ds(row0, _CHUNK), :], pos_bufs[c % 2],
                pos_sems[c % 2]))
            return hs

        in_h = {0: issue_in(0)}
        out_h = {}
        for c in range(n_chunks):
            r = c % _NSETS
            row0 = seq0 + c * _CHUNK
            if c >= 2:
                for h in out_h[c - 2]:
                    h.wait()
            if c + 1 < n_chunks:
                in_h[c + 1] = issue_in(c + 1)
            for h in in_h[c]:
                h.wait()

            bset = bufs[r]
            pos_b = pos_bufs[c % 2]

            @plsc.parallel_loop(0, _CHUNK, 1)
            def row_body(rr, bset=bset, pos_b=pos_b):
                @plsc.parallel_loop(0, n_vecs, 1, unroll=4)
                def vec_body(v):
                    sl = pl.ds(v * _LANES, _LANES)
                    pv = pos_b[rr, sl]
                    for b in range(batch):
                        bset[b][rr, sl] = bset[b][rr, sl] + pv

            out_h[c] = [
                pltpu.async_copy(
                    bset[b], out_hbm.at[b, pl.ds(row0, _CHUNK), :],
                    out_sems[r])
                for b in range(batch)
            ]
        for c in (n_chunks - 2, n_chunks - 1):
            for h in out_h[c]:
                h.wait()

    return sc_kernel(inputs, pos_table)
